# baseline (speedup 1.0000x reference)
import jax
import jax.numpy as jnp
from jax.experimental import pallas as pl
from jax.experimental.pallas import tpu as pltpu

_HIGHEST = jax.lax.Precision.HIGHEST


def _make_fused_kernel(C, D, HW, nb):
    def _fused_kernel(x_ref, y_ref, pmat2_ref, wcat_ref, seg_ref, g_ref,
                      p2_ref, o_ref):
        u = (x_ref[...] + y_ref[...]).reshape(nb * C, HW)
        pb = jnp.dot(u, pmat2_ref[...],
                     preferred_element_type=jnp.float32)
        prod = pb * wcat_ref[...]
        s1 = jnp.dot(seg_ref[...], prod,
                     preferred_element_type=jnp.float32)
        b1f = p2_ref[2 * D + 2:2 * D + 3, 0:D]
        z = jnp.dot(s1, g_ref[...],
                    preferred_element_type=jnp.float32) + b1f
        z = jnp.maximum(z, 0.0)
        zx = jnp.dot(z, p2_ref[0:D, :], precision=_HIGHEST,
                     preferred_element_type=jnp.float32) \
            + p2_ref[2 * D:2 * D + 1, :]
        zy = jnp.dot(z, p2_ref[D:2 * D, :], precision=_HIGHEST,
                     preferred_element_type=jnp.float32) \
            + p2_ref[2 * D + 1:2 * D + 2, :]
        m = jnp.maximum(zx, zy)
        ex = jnp.exp(zx - m)
        ey = jnp.exp(zy - m)
        wxr = ex / (ex + ey)
        wxct = jnp.transpose(wxr)
        for i in range(nb):
            wxc = wxct[:, i:i + 1]
            wyc = 1.0 - wxc
            o_ref[i] = (x_ref[i] * wxc + y_ref[i] * wyc).astype(o_ref.dtype)

    return _fused_kernel


def kernel(x, y, pmat, w1f, b1f, wx, bx, wy, by):
    B, C, H, W = x.shape
    HW = H * W
    D = w1f.shape[1]
    ppd = w1f.shape[0] // C
    nb = 2 if B % 2 == 0 else 1

    xf = x.reshape(B, C, HW)
    yf = y.reshape(B, C, HW)

    pmat2 = jnp.repeat(pmat[:, :ppd], D, axis=1)
    wcat = jnp.tile(w1f.reshape(C, ppd * D), (nb, 1))
    seg = jnp.repeat(jnp.eye(nb, dtype=x.dtype), C, axis=1)
    g = jnp.tile(jnp.eye(D, dtype=x.dtype), (ppd, 1))
    p2 = jnp.concatenate(
        [wx, wy, bx.reshape(1, C), by.reshape(1, C),
         jnp.pad(b1f.reshape(1, D), ((0, 0), (0, C - D)))],
        axis=0)

    out = pl.pallas_call(
        _make_fused_kernel(C, D, HW, nb),
        out_shape=jax.ShapeDtypeStruct((B, C, HW), x.dtype),
        grid=(B // nb,),
        in_specs=[
            pl.BlockSpec((nb, C, HW), lambda b: (b, 0, 0)),
            pl.BlockSpec((nb, C, HW), lambda b: (b, 0, 0)),
            pl.BlockSpec((HW, ppd * D), lambda b: (0, 0)),
            pl.BlockSpec((nb * C, ppd * D), lambda b: (0, 0)),
            pl.BlockSpec((nb, nb * C), lambda b: (0, 0)),
            pl.BlockSpec((ppd * D, D), lambda b: (0, 0)),
            pl.BlockSpec((2 * D + 3, C), lambda b: (0, 0)),
        ],
        out_specs=pl.BlockSpec((nb, C, HW), lambda b: (b, 0, 0)),
        compiler_params=pltpu.CompilerParams(
            dimension_semantics=("parallel",),
            vmem_limit_bytes=48 << 20),
    )(xf, yf, pmat2, wcat, seg, g, p2)

    return out.reshape(B, C, H, W), None, y


# speedup vs baseline: 1.0287x; 1.0287x over previous
import jax
import jax.numpy as jnp
from jax.experimental import pallas as pl
from jax.experimental.pallas import tpu as pltpu

_HIGHEST = jax.lax.Precision.HIGHEST


def _make_fused_kernel(C, D, HW, nb):
    def _fused_kernel(x_ref, y_ref, pmat2_ref, wcat_ref, p2_ref, o_ref):
        ppdD = wcat_ref.shape[1]
        u = (x_ref[...] + y_ref[...]).reshape(nb * C, HW)
        pb = jnp.dot(u, pmat2_ref[...],
                     preferred_element_type=jnp.float32)
        prod = pb * wcat_ref[...]
        s1 = jnp.sum(prod.reshape(nb, C, ppdD), axis=1)
        b1f = p2_ref[2 * D + 2:2 * D + 3, 0:D]
        z = jnp.sum(s1.reshape(nb, ppdD // D, D), axis=1) + b1f
        z = jnp.maximum(z, 0.0)
        zx = jnp.dot(z, p2_ref[0:D, :], precision=_HIGHEST,
                     preferred_element_type=jnp.float32) \
            + p2_ref[2 * D:2 * D + 1, :]
        zy = jnp.dot(z, p2_ref[D:2 * D, :], precision=_HIGHEST,
                     preferred_element_type=jnp.float32) \
            + p2_ref[2 * D + 1:2 * D + 2, :]
        m = jnp.maximum(zx, zy)
        ex = jnp.exp(zx - m)
        ey = jnp.exp(zy - m)
        wxr = ex / (ex + ey)
        wxct = jnp.transpose(wxr)
        for i in range(nb):
            wxc = wxct[:, i:i + 1]
            wyc = 1.0 - wxc
            o_ref[i] = (x_ref[i] * wxc + y_ref[i] * wyc).astype(o_ref.dtype)

    return _fused_kernel


def kernel(x, y, pmat, w1f, b1f, wx, bx, wy, by):
    B, C, H, W = x.shape
    HW = H * W
    D = w1f.shape[1]
    ppd = w1f.shape[0] // C
    nb = 4 if B % 4 == 0 else 1

    xf = x.reshape(B, C, HW)
    yf = y.reshape(B, C, HW)

    pmat2 = jnp.repeat(pmat[:, :ppd], D, axis=1)
    wcat = jnp.tile(w1f.reshape(C, ppd * D), (nb, 1))
    p2 = jnp.concatenate(
        [wx, wy, bx.reshape(1, C), by.reshape(1, C),
         jnp.pad(b1f.reshape(1, D), ((0, 0), (0, C - D)))],
        axis=0)

    out = pl.pallas_call(
        _make_fused_kernel(C, D, HW, nb),
        out_shape=jax.ShapeDtypeStruct((B, C, HW), x.dtype),
        grid=(B // nb,),
        in_specs=[
            pl.BlockSpec((nb, C, HW), lambda b: (b, 0, 0)),
            pl.BlockSpec((nb, C, HW), lambda b: (b, 0, 0)),
            pl.BlockSpec((HW, ppd * D), lambda b: (0, 0)),
            pl.BlockSpec((nb * C, ppd * D), lambda b: (0, 0)),
            pl.BlockSpec((2 * D + 3, C), lambda b: (0, 0)),
        ],
        out_specs=pl.BlockSpec((nb, C, HW), lambda b: (b, 0, 0)),
        compiler_params=pltpu.CompilerParams(
            dimension_semantics=("parallel",),
            vmem_limit_bytes=48 << 20),
    )(xf, yf, pmat2, wcat, p2)

    return out.reshape(B, C, H, W), None, y
